# baseline (speedup 1.0000x reference)
import functools

import jax
import jax.numpy as jnp
from jax import lax
from jax.experimental import pallas as pl
from jax.experimental.pallas import tpu as pltpu
from jax.experimental.pallas import tpu_sc as plsc

VOCAB1 = 100001
EMBED_DIM = 8
BATCH = 16384


@functools.lru_cache(maxsize=None)
def _make_sc_gather(V, D, B):
    info = plsc.get_sparse_core_info()
    NC, NS = info.num_cores, info.num_subcores
    NW = NC * NS
    b_per_w = B // NW
    mesh = plsc.VectorSubcoreMesh(core_axis_name="c", subcore_axis_name="s")

    @functools.partial(
        pl.kernel,
        mesh=mesh,
        compiler_params=pltpu.CompilerParams(use_tc_tiling_on_sc=False),
        out_type=jax.ShapeDtypeStruct((B, D), jnp.float32),
        scratch_types=[
            pltpu.VMEM((b_per_w,), jnp.int32),
            pltpu.VMEM((b_per_w, D), jnp.float32),
            pltpu.SemaphoreType.DMA,
        ],
    )
    def gather(table_hbm, idx_hbm, out_hbm, idx_v, rows_v, sem):
        wid = lax.axis_index("s") * NC + lax.axis_index("c")
        base = wid * b_per_w
        pltpu.sync_copy(idx_hbm.at[pl.ds(base, b_per_w)], idx_v)
        pltpu.async_copy(table_hbm.at[idx_v], rows_v, sem).wait()
        pltpu.sync_copy(rows_v, out_hbm.at[pl.ds(base, b_per_w)])

    return gather


def _mlp_body(emb_ref, w1_ref, b1_ref, w2_ref, b2_ref, out_ref):
    emb = emb_ref[...].astype(jnp.bfloat16)
    w1 = w1_ref[...].astype(jnp.bfloat16)
    h = jnp.dot(emb, w1, preferred_element_type=jnp.float32)
    h = jnp.maximum(h + b1_ref[...], 0.0)
    w2 = w2_ref[...].astype(jnp.bfloat16)
    out = jnp.dot(h.astype(jnp.bfloat16), w2, preferred_element_type=jnp.float32)
    out_ref[...] = out + b2_ref[...]


def _mlp(emb, W1, b1, W2, b2, tile=4096):
    B = emb.shape[0]
    H = W1.shape[1]
    O = W2.shape[1]
    grid = (B // tile,)
    return pl.pallas_call(
        _mlp_body,
        grid=grid,
        in_specs=[
            pl.BlockSpec((tile, EMBED_DIM), lambda i: (i, 0)),
            pl.BlockSpec((EMBED_DIM, H), lambda i: (0, 0)),
            pl.BlockSpec((1, H), lambda i: (0, 0)),
            pl.BlockSpec((H, O), lambda i: (0, 0)),
            pl.BlockSpec((1, O), lambda i: (0, 0)),
        ],
        out_specs=pl.BlockSpec((tile, O), lambda i: (i, 0)),
        out_shape=jax.ShapeDtypeStruct((B, O), jnp.float32),
    )(emb, W1, b1.reshape(1, H), W2, b2.reshape(1, O))


def kernel(indices, table, W1, b1, W2, b2):
    emb = _make_sc_gather(VOCAB1, EMBED_DIM, BATCH)(table, indices)
    return _mlp(emb, W1, b1, W2, b2)


# speedup vs baseline: 2.7830x; 2.7830x over previous
import functools

import jax
import jax.numpy as jnp
from jax import lax
from jax.experimental import pallas as pl
from jax.experimental.pallas import tpu as pltpu
from jax.experimental.pallas import tpu_sc as plsc

VOCAB1 = 100001
VPAD = 100008
EMBED_DIM = 8
BATCH = 16384


@functools.lru_cache(maxsize=None)
def _make_sc_gather(vpad, D, B):
    info = plsc.get_sparse_core_info()
    NC, NS = info.num_cores, info.num_subcores
    NW = NC * NS
    b_per_w = B // NW
    mesh = plsc.VectorSubcoreMesh(core_axis_name="c", subcore_axis_name="s")

    @functools.partial(
        pl.kernel,
        mesh=mesh,
        compiler_params=pltpu.CompilerParams(use_tc_tiling_on_sc=False),
        out_type=jax.ShapeDtypeStruct((D, B), jnp.float32),
        scratch_types=[
            pltpu.VMEM((b_per_w,), jnp.int32),
            pltpu.VMEM((D, b_per_w), jnp.float32),
            pltpu.SemaphoreType.DMA,
        ],
    )
    def gather(tflat_hbm, idx_hbm, out_hbm, idx_v, rows_v, sem):
        wid = lax.axis_index("s") * NC + lax.axis_index("c")
        base = wid * b_per_w
        pltpu.sync_copy(idx_hbm.at[pl.ds(base, b_per_w)], idx_v)
        descs = [
            pltpu.async_copy(
                tflat_hbm.at[pl.ds(d * vpad, vpad)].at[idx_v],
                rows_v.at[d],
                sem,
            )
            for d in range(D)
        ]
        for desc in descs:
            desc.wait()
        pltpu.sync_copy(rows_v, out_hbm.at[:, pl.ds(base, b_per_w)])

    return gather


def _mlp_t_body(embt_ref, w1_ref, b1_ref, w2_ref, b2_ref, out_ref):
    embt = embt_ref[...].astype(jnp.bfloat16)
    w1 = w1_ref[...].astype(jnp.bfloat16)
    ht = lax.dot_general(w1, embt, (((0,), (0,)), ((), ())),
                         preferred_element_type=jnp.float32)
    ht = jnp.maximum(ht + b1_ref[...], 0.0)
    w2 = w2_ref[...].astype(jnp.bfloat16)
    outt = lax.dot_general(w2, ht.astype(jnp.bfloat16), (((0,), (0,)), ((), ())),
                           preferred_element_type=jnp.float32)
    out_ref[...] = outt + b2_ref[...]


def _mlp_t(embt, W1, b1, W2, b2, tile=4096):
    B = embt.shape[1]
    H = W1.shape[1]
    O = W2.shape[1]
    return pl.pallas_call(
        _mlp_t_body,
        grid=(B // tile,),
        in_specs=[
            pl.BlockSpec((EMBED_DIM, tile), lambda i: (0, i)),
            pl.BlockSpec((EMBED_DIM, H), lambda i: (0, 0)),
            pl.BlockSpec((H, 1), lambda i: (0, 0)),
            pl.BlockSpec((H, O), lambda i: (0, 0)),
            pl.BlockSpec((O, 1), lambda i: (0, 0)),
        ],
        out_specs=pl.BlockSpec((O, tile), lambda i: (0, i)),
        out_shape=jax.ShapeDtypeStruct((O, B), jnp.float32),
    )(embt, W1, b1.reshape(H, 1), W2, b2.reshape(O, 1))


def kernel(indices, table, W1, b1, W2, b2):
    tflat = jnp.pad(table.T, ((0, 0), (0, VPAD - VOCAB1))).reshape(-1)
    embt = _make_sc_gather(VPAD, EMBED_DIM, BATCH)(tflat, indices)
    outt = _mlp_t(embt, W1, b1, W2, b2)
    return outt.T
